# baseline (speedup 1.0000x reference)
import functools

import jax
import jax.numpy as jnp
from jax import lax
from jax.experimental import pallas as pl
from jax.experimental.pallas import tpu as pltpu
from jax.experimental.pallas import tpu_sc as plsc

_B, _L = 4096, 200
_EMB = 64
_ND = 10
_PAD = 0
_NC, _NS, _LANES = 2, 16, 16
_NW = _NC * _NS
_RPW = _B // _NW
_IPW = _RPW * _L
_SPLIT = 120


def _sc_pool_sums(emb, flat_tok):
    mesh = plsc.VectorSubcoreMesh(core_axis_name="c", subcore_axis_name="s")

    @functools.partial(
        pl.kernel,
        mesh=mesh,
        out_type=jax.ShapeDtypeStruct((_B, _EMB), jnp.float32),
        compiler_params=pltpu.CompilerParams(use_tc_tiling_on_sc=False),
        scratch_types=[
            pltpu.VMEM((_IPW,), jnp.int32),
            pltpu.VMEM((_L, _EMB), jnp.float32),
            pltpu.VMEM((_RPW, _EMB), jnp.float32),
            pltpu.SemaphoreType.DMA,
        ],
    )
    def k(emb_hbm, tok_hbm, out_hbm, idx_v, buf, acc, sem):
        wid = lax.axis_index("s") * _NC + lax.axis_index("c")
        base = wid * _RPW
        pltpu.sync_copy(tok_hbm.at[pl.ds(base * _L, _IPW)], idx_v)

        zero = jnp.zeros((_LANES,), jnp.float32)

        @pl.loop(0, _RPW)
        def _(i):
            for c in range(_EMB // _LANES):
                acc[i, pl.ds(c * _LANES, _LANES)] = zero

        @pl.loop(0, _RPW)
        def _(r):
            off = r * _L
            c1 = pltpu.async_copy(
                emb_hbm.at[idx_v.at[pl.ds(off, _SPLIT)]],
                buf.at[pl.ds(0, _SPLIT)], sem)
            c2 = pltpu.async_copy(
                emb_hbm.at[idx_v.at[pl.ds(off + _SPLIT, _L - _SPLIT)]],
                buf.at[pl.ds(_SPLIT, _L - _SPLIT)], sem)
            c1.wait()
            c2.wait()

            @pl.loop(0, _L, step=8)
            def _(t):
                for c in range(_EMB // _LANES):
                    sl = pl.ds(c * _LANES, _LANES)
                    s = buf[t, sl]
                    for u in range(1, 8):
                        s = s + buf[t + u, sl]
                    acc[r, sl] += s

        pltpu.sync_copy(acc, out_hbm.at[pl.ds(base, _RPW)])

    return k(emb, flat_tok)


def _tc_head(tokens, sums, emb0, W1, b1, W2, b2):
    def body(tok_ref, sums_ref, emb0_ref, W1_ref, b1_ref, W2_ref, b2_ref,
             out_ref):
        cnt = jnp.sum((tok_ref[...] != _PAD).astype(jnp.float32), axis=1,
                      keepdims=True)
        npad = jnp.float32(_L) - cnt
        avg = (sums_ref[...] - npad * emb0_ref[...]) / cnt
        h = jnp.maximum(
            jnp.dot(avg, W1_ref[...], preferred_element_type=jnp.float32)
            + b1_ref[...], 0.0)
        out_ref[...] = (
            jnp.dot(h, W2_ref[...], preferred_element_type=jnp.float32)
            + b2_ref[...])

    return pl.pallas_call(
        body,
        out_shape=jax.ShapeDtypeStruct((_B, W2.shape[1]), jnp.float32),
    )(tokens, sums, emb0, W1, b1, W2, b2)


def kernel(tokens, emb, W1, b1, W2, b2):
    flat_tok = tokens.reshape(-1)
    sums = _sc_pool_sums(emb, flat_tok)
    out = _tc_head(tokens, sums, emb[0:1], W1, b1.reshape(1, -1), W2,
                   b2.reshape(1, -1))
    return out.reshape(_B, -1, _ND)


# speedup vs baseline: 1.1786x; 1.1786x over previous
import functools

import jax
import jax.numpy as jnp
from jax import lax
from jax.experimental import pallas as pl
from jax.experimental.pallas import tpu as pltpu
from jax.experimental.pallas import tpu_sc as plsc

_B, _L = 4096, 200
_EMB = 64
_ND = 10
_PAD = 0
_NC, _NS, _LANES = 2, 16, 16
_NW = _NC * _NS
_RPW = _B // _NW
_IPW = _RPW * _L
_SPLIT = 120
_NBUF = 4


def _sc_pool_sums(emb, flat_tok):
    mesh = plsc.VectorSubcoreMesh(core_axis_name="c", subcore_axis_name="s")

    @functools.partial(
        pl.kernel,
        mesh=mesh,
        out_type=jax.ShapeDtypeStruct((_B, _EMB), jnp.float32),
        compiler_params=pltpu.CompilerParams(use_tc_tiling_on_sc=False),
        scratch_types=[
            pltpu.VMEM((_IPW,), jnp.int32),
            pltpu.VMEM((_NBUF, _L, _EMB), jnp.float32),
            pltpu.VMEM((_RPW, _EMB), jnp.float32),
        ] + [pltpu.SemaphoreType.DMA] * _NBUF,
    )
    def k(emb_hbm, tok_hbm, out_hbm, idx_v, bufs, acc, *sems):
        wid = lax.axis_index("s") * _NC + lax.axis_index("c")
        base = wid * _RPW
        pltpu.sync_copy(tok_hbm.at[pl.ds(base * _L, _IPW)], idx_v)

        zero = jnp.zeros((_LANES,), jnp.float32)

        @pl.loop(0, _RPW)
        def _(i):
            for c in range(_EMB // _LANES):
                acc[i, pl.ds(c * _LANES, _LANES)] = zero

        def fire(row, b):
            off = row * _L
            pltpu.async_copy(
                emb_hbm.at[idx_v.at[pl.ds(off, _SPLIT)]],
                bufs.at[b, pl.ds(0, _SPLIT)], sems[b])
            pltpu.async_copy(
                emb_hbm.at[idx_v.at[pl.ds(off + _SPLIT, _L - _SPLIT)]],
                bufs.at[b, pl.ds(_SPLIT, _L - _SPLIT)], sems[b])

        def drain(b):
            pltpu.make_async_copy(
                out_hbm.at[pl.ds(0, _L)], bufs.at[b], sems[b]).wait()

        for b in range(_NBUF):
            fire(b, b)

        @pl.loop(0, _RPW, step=_NBUF)
        def _(g):
            for b in range(_NBUF):
                row = g + b
                drain(b)

                @pl.loop(0, _L, step=8)
                def _(t):
                    for c in range(_EMB // _LANES):
                        sl = pl.ds(c * _LANES, _LANES)
                        s = bufs[b, t, sl]
                        for u in range(1, 8):
                            s = s + bufs[b, t + u, sl]
                        acc[row, sl] += s

                nxt = row + _NBUF

                @pl.when(nxt < _RPW)
                def _():
                    fire(nxt, b)

        pltpu.sync_copy(acc, out_hbm.at[pl.ds(base, _RPW)])

    return k(emb, flat_tok)


def _tc_head(tokens, sums, emb0, W1, b1, W2, b2):
    def body(tok_ref, sums_ref, emb0_ref, W1_ref, b1_ref, W2_ref, b2_ref,
             out_ref):
        cnt = jnp.sum((tok_ref[...] != _PAD).astype(jnp.float32), axis=1,
                      keepdims=True)
        npad = jnp.float32(_L) - cnt
        avg = (sums_ref[...] - npad * emb0_ref[...]) / cnt
        h = jnp.maximum(
            jnp.dot(avg, W1_ref[...], preferred_element_type=jnp.float32)
            + b1_ref[...], 0.0)
        out_ref[...] = (
            jnp.dot(h, W2_ref[...], preferred_element_type=jnp.float32)
            + b2_ref[...])

    return pl.pallas_call(
        body,
        out_shape=jax.ShapeDtypeStruct((_B, W2.shape[1]), jnp.float32),
    )(tokens, sums, emb0, W1, b1, W2, b2)


def kernel(tokens, emb, W1, b1, W2, b2):
    flat_tok = tokens.reshape(-1)
    sums = _sc_pool_sums(emb, flat_tok)
    out = _tc_head(tokens, sums, emb[0:1], W1, b1.reshape(1, -1), W2,
                   b2.reshape(1, -1))
    return out.reshape(_B, -1, _ND)


# speedup vs baseline: 1.7069x; 1.4481x over previous
import functools

import jax
import jax.numpy as jnp
from jax import lax
from jax.experimental import pallas as pl
from jax.experimental.pallas import tpu as pltpu
from jax.experimental.pallas import tpu_sc as plsc

_B, _L = 4096, 200
_EMB = 64
_ND = 10
_PAD = 0
_NC, _NS, _LANES = 2, 16, 16
_NW = _NC * _NS
_RPW = _B // _NW
_IPW = _RPW * _L
_SPLIT = 120
_NBUF = 4


def _sc_pool_sums(emb, flat_tok):
    mesh = plsc.VectorSubcoreMesh(core_axis_name="c", subcore_axis_name="s")

    @functools.partial(
        pl.kernel,
        mesh=mesh,
        out_type=jax.ShapeDtypeStruct((_B, _EMB), jnp.float32),
        compiler_params=pltpu.CompilerParams(use_tc_tiling_on_sc=False),
        scratch_types=[
            pltpu.VMEM((_IPW,), jnp.int32),
            pltpu.VMEM((_NBUF, _L, _EMB), jnp.float32),
            pltpu.VMEM((_RPW, _EMB), jnp.float32),
        ] + [pltpu.SemaphoreType.DMA] * _NBUF,
    )
    def k(emb_hbm, tok_hbm, out_hbm, idx_v, bufs, acc, *sems):
        wid = lax.axis_index("s") * _NC + lax.axis_index("c")
        base = wid * _RPW
        pltpu.sync_copy(tok_hbm.at[pl.ds(base * _L, _IPW)], idx_v)

        zero = jnp.zeros((_LANES,), jnp.float32)

        @pl.loop(0, _RPW)
        def _(i):
            for c in range(_EMB // _LANES):
                acc[i, pl.ds(c * _LANES, _LANES)] = zero

        def fire(row, b):
            off = row * _L
            pltpu.async_copy(
                emb_hbm.at[idx_v.at[pl.ds(off, _SPLIT)]],
                bufs.at[b, pl.ds(0, _SPLIT)], sems[b])
            pltpu.async_copy(
                emb_hbm.at[idx_v.at[pl.ds(off + _SPLIT, _L - _SPLIT)]],
                bufs.at[b, pl.ds(_SPLIT, _L - _SPLIT)], sems[b])

        def drain(b):
            pltpu.make_async_copy(
                out_hbm.at[pl.ds(0, _L)], bufs.at[b], sems[b]).wait()

        for b in range(_NBUF):
            fire(b, b)

        @pl.loop(0, _RPW, step=_NBUF)
        def _(g):
            for b in range(_NBUF):
                row = g + b
                drain(b)

                @pl.loop(0, _L, step=8)
                def _(t):
                    for c in range(_EMB // _LANES):
                        sl = pl.ds(c * _LANES, _LANES)
                        s = bufs[b, t, sl]
                        for u in range(1, 8):
                            s = s + bufs[b, t + u, sl]
                        acc[row, sl] += s

                nxt = row + _NBUF

                @pl.when(nxt < _RPW)
                def _():
                    fire(nxt, b)

        pltpu.sync_copy(acc, out_hbm.at[pl.ds(base, _RPW)])

    return k(emb, flat_tok)


_TCOL = 4096


def _tc_repack(embT):
    def body(in_ref, out_ref):
        out_ref[:, 0:_EMB] = in_ref[...].T

    vocab = embT.shape[1]
    nblk = pl.cdiv(vocab, _TCOL)
    return pl.pallas_call(
        body,
        grid=(nblk,),
        in_specs=[pl.BlockSpec((_EMB, _TCOL), lambda j: (0, j))],
        out_specs=pl.BlockSpec((_TCOL, 2 * _EMB), lambda j: (j, 0)),
        out_shape=jax.ShapeDtypeStruct((nblk * _TCOL, 2 * _EMB),
                                       jnp.float32),
    )(embT)


def _tc_head(tokens, sums, emb0, W1, b1, W2, b2):
    def body(tok_ref, sums_ref, emb0_ref, W1_ref, b1_ref, W2_ref, b2_ref,
             out_ref):
        cnt = jnp.sum((tok_ref[...] != _PAD).astype(jnp.float32), axis=1,
                      keepdims=True)
        npad = jnp.float32(_L) - cnt
        avg = (sums_ref[...] - npad * emb0_ref[...]) / cnt
        h = jnp.maximum(
            jnp.dot(avg, W1_ref[...], preferred_element_type=jnp.float32)
            + b1_ref[...], 0.0)
        out_ref[...] = (
            jnp.dot(h, W2_ref[...], preferred_element_type=jnp.float32)
            + b2_ref[...])

    return pl.pallas_call(
        body,
        out_shape=jax.ShapeDtypeStruct((_B, W2.shape[1]), jnp.float32),
    )(tokens, sums, emb0, W1, b1, W2, b2)


def kernel(tokens, emb, W1, b1, W2, b2):
    flat_tok = (tokens * 2).reshape(-1)
    packed = _tc_repack(emb.T)
    emb_lin = packed.reshape(packed.shape[0] * 2, _EMB)
    sums = _sc_pool_sums(emb_lin, flat_tok)
    out = _tc_head(tokens, sums, emb[0:1], W1, b1.reshape(1, -1), W2,
                   b2.reshape(1, -1))
    return out.reshape(_B, -1, _ND)


# speedup vs baseline: 1.7525x; 1.0268x over previous
import functools

import jax
import jax.numpy as jnp
from jax import lax
from jax.experimental import pallas as pl
from jax.experimental.pallas import tpu as pltpu
from jax.experimental.pallas import tpu_sc as plsc

_B, _L = 4096, 200
_EMB = 64
_ND = 10
_PAD = 0
_NC, _NS, _LANES = 2, 16, 16
_NW = _NC * _NS
_RPW = _B // _NW
_IPW = _RPW * _L
_SPLIT = 120
_NBUF = 4


def _sc_pool_sums(emb, flat_tok):
    mesh = plsc.VectorSubcoreMesh(core_axis_name="c", subcore_axis_name="s")

    @functools.partial(
        pl.kernel,
        mesh=mesh,
        out_type=jax.ShapeDtypeStruct((_B, _EMB), jnp.float32),
        compiler_params=pltpu.CompilerParams(use_tc_tiling_on_sc=False),
        scratch_types=[
            pltpu.VMEM((_IPW,), jnp.int32),
            pltpu.VMEM((_NBUF, _L, _EMB), jnp.float32),
            pltpu.VMEM((_RPW, _EMB), jnp.float32),
        ] + [pltpu.SemaphoreType.DMA] * _NBUF,
    )
    def k(emb_hbm, tok_hbm, out_hbm, idx_v, bufs, acc, *sems):
        wid = lax.axis_index("s") * _NC + lax.axis_index("c")
        base = wid * _RPW
        pltpu.sync_copy(tok_hbm.at[pl.ds(base * _L, _IPW)], idx_v)

        zero = jnp.zeros((_LANES,), jnp.float32)

        @pl.loop(0, _RPW)
        def _(i):
            for c in range(_EMB // _LANES):
                acc[i, pl.ds(c * _LANES, _LANES)] = zero

        def fire(row, b):
            off = row * _L
            pltpu.async_copy(
                emb_hbm.at[idx_v.at[pl.ds(off, _SPLIT)]],
                bufs.at[b, pl.ds(0, _SPLIT)], sems[b])
            pltpu.async_copy(
                emb_hbm.at[idx_v.at[pl.ds(off + _SPLIT, _L - _SPLIT)]],
                bufs.at[b, pl.ds(_SPLIT, _L - _SPLIT)], sems[b])

        def drain(b):
            pltpu.make_async_copy(
                out_hbm.at[pl.ds(0, _L)], bufs.at[b], sems[b]).wait()

        for b in range(_NBUF):
            fire(b, b)

        @pl.loop(0, _RPW, step=_NBUF)
        def _(g):
            for b in range(_NBUF):
                row = g + b
                drain(b)

                @pl.loop(0, _L, step=8)
                def _(t):
                    for c in range(_EMB // _LANES):
                        sl = pl.ds(c * _LANES, _LANES)
                        s = bufs[b, t, sl]
                        for u in range(1, 8):
                            s = s + bufs[b, t + u, sl]
                        acc[row, sl] += s

                nxt = row + _NBUF

                @pl.when(nxt < _RPW)
                def _():
                    fire(nxt, b)

        pltpu.sync_copy(acc, out_hbm.at[pl.ds(base, _RPW)])

    return k(emb, flat_tok)


_TCOL = 4096


def _tc_repack(embT):
    def body(in_ref, out_ref):
        xt = in_ref[...].T
        out_ref[:, 0:_EMB] = xt[0:_TCOL // 2, :]
        out_ref[:, _EMB:2 * _EMB] = xt[_TCOL // 2:_TCOL, :]

    vocab = embT.shape[1]
    nblk = pl.cdiv(vocab, _TCOL)
    return pl.pallas_call(
        body,
        grid=(nblk,),
        in_specs=[pl.BlockSpec((_EMB, _TCOL), lambda j: (0, j))],
        out_specs=pl.BlockSpec((_TCOL // 2, 2 * _EMB), lambda j: (j, 0)),
        out_shape=jax.ShapeDtypeStruct((nblk * _TCOL // 2, 2 * _EMB),
                                       jnp.float32),
    )(embT)


def _tc_head(tokens, sums, emb0, W1, b1, W2, b2):
    def body(tok_ref, sums_ref, emb0_ref, W1_ref, b1_ref, W2_ref, b2_ref,
             out_ref):
        cnt = jnp.sum((tok_ref[...] != _PAD).astype(jnp.float32), axis=1,
                      keepdims=True)
        npad = jnp.float32(_L) - cnt
        avg = (sums_ref[...] - npad * emb0_ref[...]) / cnt
        h = jnp.maximum(
            jnp.dot(avg, W1_ref[...], preferred_element_type=jnp.float32)
            + b1_ref[...], 0.0)
        out_ref[...] = (
            jnp.dot(h, W2_ref[...], preferred_element_type=jnp.float32)
            + b2_ref[...])

    return pl.pallas_call(
        body,
        out_shape=jax.ShapeDtypeStruct((_B, W2.shape[1]), jnp.float32),
    )(tokens, sums, emb0, W1, b1, W2, b2)


def kernel(tokens, emb, W1, b1, W2, b2):
    remapped = ((tokens & ~4095) + ((tokens & 2047) << 1)
                + ((tokens >> 11) & 1))
    flat_tok = remapped.reshape(-1)
    packed = _tc_repack(emb.T)
    emb_lin = packed.reshape(packed.shape[0] * 2, _EMB)
    sums = _sc_pool_sums(emb_lin, flat_tok)
    out = _tc_head(tokens, sums, emb[0:1], W1, b1.reshape(1, -1), W2,
                   b2.reshape(1, -1))
    return out.reshape(_B, -1, _ND)


# speedup vs baseline: 2.1400x; 1.2211x over previous
import functools

import jax
import jax.numpy as jnp
from jax import lax
from jax.experimental import pallas as pl
from jax.experimental.pallas import tpu as pltpu
from jax.experimental.pallas import tpu_sc as plsc

_B, _L = 4096, 200
_EMB = 64
_ND = 10
_PAD = 0
_NC, _NS, _LANES = 2, 16, 16
_NW = _NC * _NS
_RPW = _B // _NW
_IPW = _RPW * _L
_SPLIT = 120
_NBUF = 4


def _sc_pool_sums(emb, flat_tok):
    mesh = plsc.VectorSubcoreMesh(core_axis_name="c", subcore_axis_name="s")

    @functools.partial(
        pl.kernel,
        mesh=mesh,
        out_type=jax.ShapeDtypeStruct((_B, _EMB), jnp.float32),
        compiler_params=pltpu.CompilerParams(use_tc_tiling_on_sc=False),
        scratch_types=[
            pltpu.VMEM((_IPW,), jnp.int32),
            pltpu.VMEM((_NBUF, _L, _EMB), jnp.float32),
            pltpu.VMEM((_RPW, _EMB), jnp.float32),
        ] + [pltpu.SemaphoreType.DMA] * _NBUF,
    )
    def k(emb_hbm, tok_hbm, out_hbm, idx_v, bufs, acc, *sems):
        wid = lax.axis_index("s") * _NC + lax.axis_index("c")
        base = wid * _RPW
        pltpu.sync_copy(tok_hbm.at[pl.ds(base * _L, _IPW)], idx_v)

        zero = jnp.zeros((_LANES,), jnp.float32)

        @pl.loop(0, _RPW)
        def _(i):
            for c in range(_EMB // _LANES):
                acc[i, pl.ds(c * _LANES, _LANES)] = zero

        def fire(row, b):
            off = row * _L
            pltpu.async_copy(
                emb_hbm.at[idx_v.at[pl.ds(off, _SPLIT)]],
                bufs.at[b, pl.ds(0, _SPLIT)], sems[b])
            pltpu.async_copy(
                emb_hbm.at[idx_v.at[pl.ds(off + _SPLIT, _L - _SPLIT)]],
                bufs.at[b, pl.ds(_SPLIT, _L - _SPLIT)], sems[b])

        def drain(b):
            pltpu.make_async_copy(
                out_hbm.at[pl.ds(0, _L)], bufs.at[b], sems[b]).wait()

        for b in range(_NBUF):
            fire(b, b)

        @pl.loop(0, _RPW, step=_NBUF)
        def _(g):
            for b in range(_NBUF):
                row = g + b
                drain(b)

                @pl.loop(0, _L, step=8)
                def _(t):
                    for c in range(_EMB // _LANES):
                        sl = pl.ds(c * _LANES, _LANES)
                        s = bufs[b, t, sl]
                        for u in range(1, 8):
                            s = s + bufs[b, t + u, sl]
                        acc[row, sl] += s

                nxt = row + _NBUF

                @pl.when(nxt < _RPW)
                def _():
                    fire(nxt, b)

        pltpu.sync_copy(acc, out_hbm.at[pl.ds(base, _RPW)])

    return k(emb, flat_tok)


_TCOL = 16384


def _tc_repack(embT):
    def body(in_ref, out_ref):
        xt = in_ref[...].T
        out_ref[:, 0:_EMB] = xt[0:_TCOL // 2, :]
        out_ref[:, _EMB:2 * _EMB] = xt[_TCOL // 2:_TCOL, :]

    vocab = embT.shape[1]
    nblk = pl.cdiv(vocab, _TCOL)
    return pl.pallas_call(
        body,
        grid=(nblk,),
        in_specs=[pl.BlockSpec((_EMB, _TCOL), lambda j: (0, j))],
        out_specs=pl.BlockSpec((_TCOL // 2, 2 * _EMB), lambda j: (j, 0)),
        out_shape=jax.ShapeDtypeStruct((nblk * _TCOL // 2, 2 * _EMB),
                                       jnp.float32),
    )(embT)


def _tc_head(tokens, sums, emb0, W1, b1, W2, b2):
    def body(tok_ref, sums_ref, emb0_ref, W1_ref, b1_ref, W2_ref, b2_ref,
             out_ref):
        cnt = jnp.sum((tok_ref[...] != _PAD).astype(jnp.float32), axis=1,
                      keepdims=True)
        npad = jnp.float32(_L) - cnt
        avg = (sums_ref[...] - npad * emb0_ref[...]) / cnt
        h = jnp.maximum(
            jnp.dot(avg, W1_ref[...], preferred_element_type=jnp.float32)
            + b1_ref[...], 0.0)
        out_ref[...] = (
            jnp.dot(h, W2_ref[...], preferred_element_type=jnp.float32)
            + b2_ref[...])

    return pl.pallas_call(
        body,
        out_shape=jax.ShapeDtypeStruct((_B, W2.shape[1]), jnp.float32),
    )(tokens, sums, emb0, W1, b1, W2, b2)


def kernel(tokens, emb, W1, b1, W2, b2):
    half = _TCOL // 2
    remapped = ((tokens & ~(_TCOL - 1)) + ((tokens & (half - 1)) << 1)
                + ((tokens // half) & 1))
    flat_tok = remapped.reshape(-1)
    packed = _tc_repack(emb.T)
    emb_lin = packed.reshape(packed.shape[0] * 2, _EMB)
    sums = _sc_pool_sums(emb_lin, flat_tok)
    out = _tc_head(tokens, sums, emb[0:1], W1, b1.reshape(1, -1), W2,
                   b2.reshape(1, -1))
    return out.reshape(_B, -1, _ND)


# speedup vs baseline: 2.2142x; 1.0346x over previous
import functools

import jax
import jax.numpy as jnp
from jax import lax
from jax.experimental import pallas as pl
from jax.experimental.pallas import tpu as pltpu
from jax.experimental.pallas import tpu_sc as plsc

_B, _L = 4096, 200
_EMB = 64
_ND = 10
_PAD = 0
_NC, _NS, _LANES = 2, 16, 16
_NW = _NC * _NS
_RPW = _B // _NW
_IPW = _RPW * _L
_SPLIT = 120
_NBUF = 4


def _sc_pool_sums(emb, flat_tok):
    mesh = plsc.VectorSubcoreMesh(core_axis_name="c", subcore_axis_name="s")

    @functools.partial(
        pl.kernel,
        mesh=mesh,
        out_type=jax.ShapeDtypeStruct((_B, _EMB), jnp.float32),
        compiler_params=pltpu.CompilerParams(use_tc_tiling_on_sc=False),
        scratch_types=[
            pltpu.VMEM((_IPW,), jnp.int32),
            pltpu.VMEM((_NBUF, _L, _EMB), jnp.float32),
            pltpu.VMEM((_RPW, _EMB), jnp.float32),
        ] + [pltpu.SemaphoreType.DMA] * _NBUF,
    )
    def k(emb_hbm, tok_hbm, out_hbm, idx_v, bufs, acc, *sems):
        wid = lax.axis_index("s") * _NC + lax.axis_index("c")
        base = wid * _RPW
        pltpu.sync_copy(tok_hbm.at[pl.ds(base * _L, _IPW)], idx_v)

        zero = jnp.zeros((_LANES,), jnp.float32)

        @pl.loop(0, _RPW)
        def _(i):
            for c in range(_EMB // _LANES):
                acc[i, pl.ds(c * _LANES, _LANES)] = zero

        def fire(row, b):
            off = row * _L
            pltpu.async_copy(
                emb_hbm.at[idx_v.at[pl.ds(off, _SPLIT)]],
                bufs.at[b, pl.ds(0, _SPLIT)], sems[b])
            pltpu.async_copy(
                emb_hbm.at[idx_v.at[pl.ds(off + _SPLIT, _L - _SPLIT)]],
                bufs.at[b, pl.ds(_SPLIT, _L - _SPLIT)], sems[b])

        def drain(b):
            pltpu.make_async_copy(
                out_hbm.at[pl.ds(0, _L)], bufs.at[b], sems[b]).wait()

        for b in range(_NBUF):
            fire(b, b)

        @pl.loop(0, _RPW, step=_NBUF)
        def _(g):
            for b in range(_NBUF):
                row = g + b
                drain(b)

                @pl.loop(0, _L, step=8)
                def _(t):
                    for c in range(_EMB // _LANES):
                        sl = pl.ds(c * _LANES, _LANES)
                        s = bufs[b, t, sl]
                        for u in range(1, 8):
                            s = s + bufs[b, t + u, sl]
                        acc[row, sl] += s

                nxt = row + _NBUF

                @pl.when(nxt < _RPW)
                def _():
                    fire(nxt, b)

        pltpu.sync_copy(acc, out_hbm.at[pl.ds(base, _RPW)])

    return k(emb, flat_tok)


_TCOL = 32768


def _tc_repack(embT):
    def body(in_ref, out_ref):
        xt = in_ref[...].T
        out_ref[:, 0:_EMB] = xt[0:_TCOL // 2, :]
        out_ref[:, _EMB:2 * _EMB] = xt[_TCOL // 2:_TCOL, :]

    vocab = embT.shape[1]
    nblk = pl.cdiv(vocab, _TCOL)
    return pl.pallas_call(
        body,
        grid=(nblk,),
        in_specs=[pl.BlockSpec((_EMB, _TCOL), lambda j: (0, j))],
        out_specs=pl.BlockSpec((_TCOL // 2, 2 * _EMB), lambda j: (j, 0)),
        out_shape=jax.ShapeDtypeStruct((nblk * _TCOL // 2, 2 * _EMB),
                                       jnp.float32),
    )(embT)


def _tc_head(tokens, sums, emb0, W1, b1, W2, b2):
    def body(tok_ref, sums_ref, emb0_ref, W1_ref, b1_ref, W2_ref, b2_ref,
             out_ref):
        cnt = jnp.sum((tok_ref[...] != _PAD).astype(jnp.float32), axis=1,
                      keepdims=True)
        npad = jnp.float32(_L) - cnt
        avg = (sums_ref[...] - npad * emb0_ref[...]) / cnt
        h = jnp.maximum(
            jnp.dot(avg, W1_ref[...], preferred_element_type=jnp.float32)
            + b1_ref[...], 0.0)
        out_ref[...] = (
            jnp.dot(h, W2_ref[...], preferred_element_type=jnp.float32)
            + b2_ref[...])

    return pl.pallas_call(
        body,
        out_shape=jax.ShapeDtypeStruct((_B, W2.shape[1]), jnp.float32),
    )(tokens, sums, emb0, W1, b1, W2, b2)


def kernel(tokens, emb, W1, b1, W2, b2):
    half = _TCOL // 2
    remapped = ((tokens & ~(_TCOL - 1)) + ((tokens & (half - 1)) << 1)
                + ((tokens // half) & 1))
    flat_tok = remapped.reshape(-1)
    packed = _tc_repack(emb.T)
    emb_lin = packed.reshape(packed.shape[0] * 2, _EMB)
    sums = _sc_pool_sums(emb_lin, flat_tok)
    out = _tc_head(tokens, sums, emb[0:1], W1, b1.reshape(1, -1), W2,
                   b2.reshape(1, -1))
    return out.reshape(_B, -1, _ND)


# speedup vs baseline: 2.2678x; 1.0242x over previous
import functools

import jax
import jax.numpy as jnp
from jax import lax
from jax.experimental import pallas as pl
from jax.experimental.pallas import tpu as pltpu
from jax.experimental.pallas import tpu_sc as plsc

_B, _L = 4096, 200
_EMB = 64
_ND = 10
_PAD = 0
_NC, _NS, _LANES = 2, 16, 16
_NW = _NC * _NS
_RPW = _B // _NW
_IPW = _RPW * _L
_SPLIT = 120
_NBUF = 4


def _sc_pool_sums(emb, flat_tok):
    mesh = plsc.VectorSubcoreMesh(core_axis_name="c", subcore_axis_name="s")

    @functools.partial(
        pl.kernel,
        mesh=mesh,
        out_type=jax.ShapeDtypeStruct((_B, _EMB), jnp.float32),
        compiler_params=pltpu.CompilerParams(use_tc_tiling_on_sc=False),
        scratch_types=[
            pltpu.VMEM((_IPW,), jnp.int32),
            pltpu.VMEM((_NBUF, _L, _EMB), jnp.float32),
            pltpu.VMEM((_RPW, _EMB), jnp.float32),
        ] + [pltpu.SemaphoreType.DMA] * _NBUF,
    )
    def k(emb_hbm, tok_hbm, out_hbm, idx_v, bufs, acc, *sems):
        wid = lax.axis_index("s") * _NC + lax.axis_index("c")
        base = wid * _RPW
        pltpu.sync_copy(tok_hbm.at[pl.ds(base * _L, _IPW)], idx_v)

        zero = jnp.zeros((_LANES,), jnp.float32)

        @pl.loop(0, _RPW)
        def _(i):
            for c in range(_EMB // _LANES):
                acc[i, pl.ds(c * _LANES, _LANES)] = zero

        def fire(row, b):
            off = row * _L
            pltpu.async_copy(
                emb_hbm.at[idx_v.at[pl.ds(off, _SPLIT)]],
                bufs.at[b, pl.ds(0, _SPLIT)], sems[b])
            pltpu.async_copy(
                emb_hbm.at[idx_v.at[pl.ds(off + _SPLIT, _L - _SPLIT)]],
                bufs.at[b, pl.ds(_SPLIT, _L - _SPLIT)], sems[b])

        def drain(b):
            pltpu.make_async_copy(
                out_hbm.at[pl.ds(0, _L)], bufs.at[b], sems[b]).wait()

        for b in range(_NBUF):
            fire(b, b)

        @pl.loop(0, _RPW, step=_NBUF)
        def _(g):
            for b in range(_NBUF):
                row = g + b
                drain(b)

                @pl.loop(0, _L, step=25)
                def _(t):
                    for c in range(_EMB // _LANES):
                        sl = pl.ds(c * _LANES, _LANES)
                        s = bufs[b, t, sl]
                        for u in range(1, 25):
                            s = s + bufs[b, t + u, sl]
                        acc[row, sl] += s

                nxt = row + _NBUF

                @pl.when(nxt < _RPW)
                def _():
                    fire(nxt, b)

        pltpu.sync_copy(acc, out_hbm.at[pl.ds(base, _RPW)])

    return k(emb, flat_tok)


_TCOL = 32768


def _tc_repack(embT):
    def body(in_ref, out_ref):
        xt = in_ref[...].T
        out_ref[:, 0:_EMB] = xt[0:_TCOL // 2, :]
        out_ref[:, _EMB:2 * _EMB] = xt[_TCOL // 2:_TCOL, :]

    vocab = embT.shape[1]
    nblk = pl.cdiv(vocab, _TCOL)
    return pl.pallas_call(
        body,
        grid=(nblk,),
        in_specs=[pl.BlockSpec((_EMB, _TCOL), lambda j: (0, j))],
        out_specs=pl.BlockSpec((_TCOL // 2, 2 * _EMB), lambda j: (j, 0)),
        out_shape=jax.ShapeDtypeStruct((nblk * _TCOL // 2, 2 * _EMB),
                                       jnp.float32),
    )(embT)


def _tc_head(tokens, sums, emb0, W1, b1, W2, b2):
    def body(tok_ref, sums_ref, emb0_ref, W1_ref, b1_ref, W2_ref, b2_ref,
             out_ref):
        cnt = jnp.sum((tok_ref[...] != _PAD).astype(jnp.float32), axis=1,
                      keepdims=True)
        npad = jnp.float32(_L) - cnt
        avg = (sums_ref[...] - npad * emb0_ref[...]) / cnt
        h = jnp.maximum(
            jnp.dot(avg, W1_ref[...], preferred_element_type=jnp.float32)
            + b1_ref[...], 0.0)
        out_ref[...] = (
            jnp.dot(h, W2_ref[...], preferred_element_type=jnp.float32)
            + b2_ref[...])

    return pl.pallas_call(
        body,
        out_shape=jax.ShapeDtypeStruct((_B, W2.shape[1]), jnp.float32),
    )(tokens, sums, emb0, W1, b1, W2, b2)


def kernel(tokens, emb, W1, b1, W2, b2):
    half = _TCOL // 2
    remapped = ((tokens & ~(_TCOL - 1)) + ((tokens & (half - 1)) << 1)
                + ((tokens // half) & 1))
    flat_tok = remapped.reshape(-1)
    packed = _tc_repack(emb.T)
    emb_lin = packed.reshape(packed.shape[0] * 2, _EMB)
    sums = _sc_pool_sums(emb_lin, flat_tok)
    out = _tc_head(tokens, sums, emb[0:1], W1, b1.reshape(1, -1), W2,
                   b2.reshape(1, -1))
    return out.reshape(_B, -1, _ND)


# speedup vs baseline: 2.5684x; 1.1326x over previous
import functools

import jax
import jax.numpy as jnp
from jax import lax
from jax.experimental import pallas as pl
from jax.experimental.pallas import tpu as pltpu
from jax.experimental.pallas import tpu_sc as plsc

_B, _L = 4096, 200
_EMB = 64
_ND = 10
_PAD = 0
_NC, _NS, _LANES = 2, 16, 16
_NW = _NC * _NS
_RPW = _B // _NW
_IPW = _RPW * _L
_SPLIT = 120
_NBUF = 4


def _sc_pool_sums(emb, flat_tok):
    mesh = plsc.VectorSubcoreMesh(core_axis_name="c", subcore_axis_name="s")

    @functools.partial(
        pl.kernel,
        mesh=mesh,
        out_type=jax.ShapeDtypeStruct((_B, _EMB), jnp.float32),
        compiler_params=pltpu.CompilerParams(use_tc_tiling_on_sc=False),
        scratch_types=[
            pltpu.VMEM((_IPW,), jnp.int32),
            pltpu.VMEM((_NBUF, _L, _EMB), jnp.float32),
            pltpu.VMEM((_RPW, _EMB), jnp.float32),
        ] + [pltpu.SemaphoreType.DMA] * _NBUF,
    )
    def k(emb_hbm, tok_hbm, out_hbm, idx_v, bufs, acc, *sems):
        wid = lax.axis_index("s") * _NC + lax.axis_index("c")
        base = wid * _RPW
        pltpu.sync_copy(tok_hbm.at[pl.ds(base * _L, _IPW)], idx_v)

        zero = jnp.zeros((_LANES,), jnp.float32)

        @pl.loop(0, _RPW)
        def _(i):
            for c in range(_EMB // _LANES):
                acc[i, pl.ds(c * _LANES, _LANES)] = zero

        def fire(row, b):
            off = row * _L
            pltpu.async_copy(
                emb_hbm.at[idx_v.at[pl.ds(off, _SPLIT)]],
                bufs.at[b, pl.ds(0, _SPLIT)], sems[b])
            pltpu.async_copy(
                emb_hbm.at[idx_v.at[pl.ds(off + _SPLIT, _L - _SPLIT)]],
                bufs.at[b, pl.ds(_SPLIT, _L - _SPLIT)], sems[b])

        def drain(b):
            pltpu.make_async_copy(
                out_hbm.at[pl.ds(0, _L)], bufs.at[b], sems[b]).wait()

        for b in range(_NBUF):
            fire(b, b)

        @pl.loop(0, _RPW, step=_NBUF)
        def _(g):
            for b in range(_NBUF):
                row = g + b
                drain(b)

                @pl.loop(0, _L, step=25)
                def _(t):
                    for c in range(_EMB // _LANES):
                        sl = pl.ds(c * _LANES, _LANES)
                        s = bufs[b, t, sl]
                        for u in range(1, 25):
                            s = s + bufs[b, t + u, sl]
                        acc[row, sl] += s

                nxt = row + _NBUF

                @pl.when(nxt < _RPW)
                def _():
                    fire(nxt, b)

        pltpu.sync_copy(acc, out_hbm.at[pl.ds(base, _RPW)])

    return k(emb, flat_tok)


_TCOL = 32768


def _tc_repack(embT):
    def body(in_ref, out_ref):
        x = in_ref[...]
        half = _TCOL // 2
        stacked = jnp.concatenate([x[:, 0:half], x[:, half:_TCOL]], axis=0)
        out_ref[...] = stacked.T

    vocab = embT.shape[1]
    nblk = pl.cdiv(vocab, _TCOL)
    return pl.pallas_call(
        body,
        grid=(nblk,),
        in_specs=[pl.BlockSpec((_EMB, _TCOL), lambda j: (0, j))],
        out_specs=pl.BlockSpec((_TCOL // 2, 2 * _EMB), lambda j: (j, 0)),
        out_shape=jax.ShapeDtypeStruct((nblk * _TCOL // 2, 2 * _EMB),
                                       jnp.float32),
    )(embT)


def _tc_head(tokens, sums, emb0, W1, b1, W2, b2):
    def body(tok_ref, sums_ref, emb0_ref, W1_ref, b1_ref, W2_ref, b2_ref,
             out_ref):
        cnt = jnp.sum((tok_ref[...] != _PAD).astype(jnp.float32), axis=1,
                      keepdims=True)
        npad = jnp.float32(_L) - cnt
        avg = (sums_ref[...] - npad * emb0_ref[...]) / cnt
        h = jnp.maximum(
            jnp.dot(avg, W1_ref[...], preferred_element_type=jnp.float32)
            + b1_ref[...], 0.0)
        out_ref[...] = (
            jnp.dot(h, W2_ref[...], preferred_element_type=jnp.float32)
            + b2_ref[...])

    return pl.pallas_call(
        body,
        out_shape=jax.ShapeDtypeStruct((_B, W2.shape[1]), jnp.float32),
    )(tokens, sums, emb0, W1, b1, W2, b2)


def kernel(tokens, emb, W1, b1, W2, b2):
    half = _TCOL // 2
    remapped = ((tokens & ~(_TCOL - 1)) + ((tokens & (half - 1)) << 1)
                + ((tokens // half) & 1))
    flat_tok = remapped.reshape(-1)
    packed = _tc_repack(emb.T)
    emb_lin = packed.reshape(packed.shape[0] * 2, _EMB)
    sums = _sc_pool_sums(emb_lin, flat_tok)
    out = _tc_head(tokens, sums, emb[0:1], W1, b1.reshape(1, -1), W2,
                   b2.reshape(1, -1))
    return out.reshape(_B, -1, _ND)


# speedup vs baseline: 3.1962x; 1.2444x over previous
import functools

import jax
import jax.numpy as jnp
from jax import lax
from jax.experimental import pallas as pl
from jax.experimental.pallas import tpu as pltpu
from jax.experimental.pallas import tpu_sc as plsc

_B, _L = 4096, 200
_EMB = 64
_ND = 10
_PAD = 0
_NC, _NS, _LANES = 2, 16, 16
_NW = _NC * _NS
_RPW = _B // _NW
_IPW = _RPW * _L
_SPLIT = 120
_NBUF = 4


def _sc_pool_sums(emb, flat_tok):
    mesh = plsc.VectorSubcoreMesh(core_axis_name="c", subcore_axis_name="s")

    @functools.partial(
        pl.kernel,
        mesh=mesh,
        out_type=jax.ShapeDtypeStruct((_B, _EMB), jnp.float32),
        compiler_params=pltpu.CompilerParams(use_tc_tiling_on_sc=False),
        scratch_types=[
            pltpu.VMEM((_IPW,), jnp.int32),
            pltpu.VMEM((_NBUF, _L, _EMB), jnp.float32),
            pltpu.VMEM((_RPW, _EMB), jnp.float32),
        ] + [pltpu.SemaphoreType.DMA] * _NBUF,
    )
    def k(emb_hbm, tok_hbm, out_hbm, idx_v, bufs, acc, *sems):
        wid = lax.axis_index("s") * _NC + lax.axis_index("c")
        base = wid * _RPW
        pltpu.sync_copy(tok_hbm.at[pl.ds(base * _L, _IPW)], idx_v)

        zero = jnp.zeros((_LANES,), jnp.float32)

        @pl.loop(0, _RPW)
        def _(i):
            for c in range(_EMB // _LANES):
                acc[i, pl.ds(c * _LANES, _LANES)] = zero

        def fire(row, b):
            off = row * _L
            pltpu.async_copy(
                emb_hbm.at[idx_v.at[pl.ds(off, _SPLIT)]],
                bufs.at[b, pl.ds(0, _SPLIT)], sems[b])
            pltpu.async_copy(
                emb_hbm.at[idx_v.at[pl.ds(off + _SPLIT, _L - _SPLIT)]],
                bufs.at[b, pl.ds(_SPLIT, _L - _SPLIT)], sems[b])

        def drain(b):
            pltpu.make_async_copy(
                out_hbm.at[pl.ds(0, _L)], bufs.at[b], sems[b]).wait()

        for b in range(_NBUF):
            fire(b, b)

        @pl.loop(0, _RPW, step=_NBUF)
        def _(g):
            for b in range(_NBUF):
                row = g + b
                drain(b)

                @pl.loop(0, _L, step=25)
                def _(t):
                    for c in range(_EMB // _LANES):
                        sl = pl.ds(c * _LANES, _LANES)
                        s0 = bufs[b, t, sl]
                        s1 = bufs[b, t + 1, sl]
                        for u in range(2, 25, 2):
                            s0 = s0 + bufs[b, t + u, sl]
                        for u in range(3, 25, 2):
                            s1 = s1 + bufs[b, t + u, sl]
                        acc[row, sl] += s0 + s1

                nxt = row + _NBUF

                @pl.when(nxt < _RPW)
                def _():
                    fire(nxt, b)

        pltpu.sync_copy(acc, out_hbm.at[pl.ds(base, _RPW)])

    return k(emb, flat_tok)


_TCOL = 32768


def _tc_repack(embT):
    def body(in_ref, out_ref):
        x = in_ref[...]
        half = _TCOL // 2
        stacked = jnp.concatenate([x[:, 0:half], x[:, half:_TCOL]], axis=0)
        out_ref[...] = stacked.T

    vocab = embT.shape[1]
    nblk = pl.cdiv(vocab, _TCOL)
    return pl.pallas_call(
        body,
        grid=(nblk,),
        in_specs=[pl.BlockSpec((_EMB, _TCOL), lambda j: (0, j))],
        out_specs=pl.BlockSpec((_TCOL // 2, 2 * _EMB), lambda j: (j, 0)),
        out_shape=jax.ShapeDtypeStruct((nblk * _TCOL // 2, 2 * _EMB),
                                       jnp.float32),
    )(embT)


def _tc_head(tokens, sums, emb0, W1, b1, W2, b2):
    def body(tok_ref, sums_ref, emb0_ref, W1_ref, b1_ref, W2_ref, b2_ref,
             out_ref):
        cnt = jnp.sum((tok_ref[...] != _PAD).astype(jnp.float32), axis=1,
                      keepdims=True)
        npad = jnp.float32(_L) - cnt
        avg = (sums_ref[...] - npad * emb0_ref[...]) / cnt
        h = jnp.maximum(
            jnp.dot(avg, W1_ref[...], preferred_element_type=jnp.float32)
            + b1_ref[...], 0.0)
        out_ref[...] = (
            jnp.dot(h, W2_ref[...], preferred_element_type=jnp.float32)
            + b2_ref[...])

    return pl.pallas_call(
        body,
        out_shape=jax.ShapeDtypeStruct((_B, W2.shape[1]), jnp.float32),
    )(tokens, sums, emb0, W1, b1, W2, b2)


def kernel(tokens, emb, W1, b1, W2, b2):
    half = _TCOL // 2
    remapped = ((tokens & ~(_TCOL - 1)) + ((tokens & (half - 1)) << 1)
                + ((tokens // half) & 1))
    flat_tok = remapped.reshape(-1)
    packed = _tc_repack(emb.T)
    emb_lin = packed.reshape(packed.shape[0] * 2, _EMB)
    sums = _sc_pool_sums(emb_lin, flat_tok)
    out = _tc_head(tokens, sums, emb[0:1], W1, b1.reshape(1, -1), W2,
                   b2.reshape(1, -1))
    return out.reshape(_B, -1, _ND)
